# Initial kernel scaffold; baseline (speedup 1.0000x reference)
import jax
import jax.numpy as jnp
from jax.experimental import pallas as pl


def kernel(hidden_states, gate_w, gate_proj, up_proj, down_proj, expert_bias):
    raise NotImplementedError("write your pallas kernel here")


# speedup vs baseline: 1.2541x; 1.2541x over previous
import functools

import jax
import jax.numpy as jnp
from jax.experimental import pallas as pl
from jax.experimental.pallas import tpu as pltpu

E = 8
EP = 128
TOP_K = 2
NEG = -1e30


def _router_body(x_ref, gw_ref, bias_ref, tokw_ref):
    x = x_ref[...]
    logits = jax.lax.dot_general(
        x, gw_ref[...], (((1,), (1,)), ((), ())),
        preferred_element_type=jnp.float32)
    rw = jax.nn.sigmoid(logits)
    scores = rw + bias_ref[...]
    idx = jax.lax.broadcasted_iota(jnp.int32, scores.shape, 1)
    m1 = jnp.max(scores, axis=1, keepdims=True)
    i1 = jnp.min(jnp.where(scores == m1, idx, EP), axis=1, keepdims=True)
    sel1 = idx == i1
    s2 = jnp.where(sel1, NEG, scores)
    m2 = jnp.max(s2, axis=1, keepdims=True)
    i2 = jnp.min(jnp.where(s2 == m2, idx, EP), axis=1, keepdims=True)
    sel = sel1 | (idx == i2)
    rw_sel = jnp.where(sel, rw, 0.0)
    rsum = jnp.sum(rw_sel, axis=1, keepdims=True)
    tokw_ref[...] = rw_sel / (rsum + 1e-6)


def _ffn_body(tokw_ref, x_ref, gp_ref, up_ref, dp_ref, out_ref):
    e = pl.program_id(1)
    x = x_ref[...]
    g = jax.lax.dot_general(x, gp_ref[0], (((1,), (1,)), ((), ())),
                            preferred_element_type=jnp.float32)
    u = jax.lax.dot_general(x, up_ref[0], (((1,), (1,)), ((), ())),
                            preferred_element_type=jnp.float32)
    h = (g * jax.nn.sigmoid(g) * u).astype(jnp.bfloat16)
    y = jax.lax.dot_general(h, dp_ref[0], (((1,), (1,)), ((), ())),
                            preferred_element_type=jnp.float32)
    eo = jax.lax.broadcasted_iota(jnp.int32, (1, EP), 1) == e
    w = jnp.sum(jnp.where(eo, tokw_ref[...], 0.0), axis=1, keepdims=True)

    @pl.when(e == 0)
    def _():
        out_ref[...] = y * w

    @pl.when(e > 0)
    def _():
        out_ref[...] += y * w


@functools.partial(jax.jit, static_argnames=("interpret",))
def kernel(hidden_states, gate_w, gate_proj, up_proj, down_proj,
           expert_bias, interpret=False):
    B, S, H = hidden_states.shape
    T = B * S
    FF = gate_proj.shape[1]
    x = hidden_states.reshape(T, H)

    gw_pad = jnp.zeros((EP, H), jnp.float32).at[:E].set(gate_w)
    bias_pad = jnp.full((1, EP), NEG, jnp.float32).at[0, :E].set(expert_bias)

    tokw = pl.pallas_call(
        _router_body,
        out_shape=jax.ShapeDtypeStruct((T, EP), jnp.float32),
        interpret=interpret,
    )(x, gw_pad, bias_pad)

    BT = 512
    xb = x.astype(jnp.bfloat16)
    gpb = gate_proj.astype(jnp.bfloat16)
    upb = up_proj.astype(jnp.bfloat16)
    dpb = down_proj.astype(jnp.bfloat16)

    out = pl.pallas_call(
        _ffn_body,
        grid=(T // BT, E),
        in_specs=[
            pl.BlockSpec((BT, EP), lambda t, e: (t, 0)),
            pl.BlockSpec((BT, H), lambda t, e: (t, 0)),
            pl.BlockSpec((1, FF, H), lambda t, e: (e, 0, 0)),
            pl.BlockSpec((1, FF, H), lambda t, e: (e, 0, 0)),
            pl.BlockSpec((1, H, FF), lambda t, e: (e, 0, 0)),
        ],
        out_specs=pl.BlockSpec((BT, H), lambda t, e: (t, 0)),
        out_shape=jax.ShapeDtypeStruct((T, H), jnp.float32),
        compiler_params=pltpu.CompilerParams(
            dimension_semantics=("parallel", "arbitrary")),
        interpret=interpret,
    )(tokw, xb, gpb, upb, dpb)

    return out.reshape(B, S, H)
